# baseline (speedup 1.0000x reference)
import functools

import jax
import jax.numpy as jnp
from jax import lax
from jax.experimental import pallas as pl
from jax.experimental.pallas import tpu as pltpu
from jax.experimental.pallas import tpu_sc as plsc

NC = 2
NS = 16
NW = NC * NS
LANES = 16

KG = 384
G = 128
KS = 128
IBG = 11
IBS = 12
EQ = 24576
HID = 64


def _silu(x):
    return x * jax.nn.sigmoid(x)


def _bd(W):
    Z = jnp.zeros_like(W)
    return jnp.concatenate(
        [jnp.concatenate([W, Z], axis=1), jnp.concatenate([Z, W], axis=1)],
        axis=0)


def _bt(b):
    return jnp.concatenate([b, b])[None, :]


def _full(spec_shape):
    return pl.BlockSpec(spec_shape, lambda i: tuple(0 for _ in spec_shape))


def _rows(bs, width):
    return pl.BlockSpec((bs, width), lambda i: (i, 0))


def _tc_embed(x12, We_p, be_t, Wa_p, Wb_p, bs=1000):
    n2 = x12.shape[0]

    def body(x_ref, we_ref, be_ref, wa_ref, wb_ref, h_ref, p_ref, q_ref):
        h = jnp.dot(x_ref[...], we_ref[...],
                    preferred_element_type=jnp.float32) + be_ref[...]
        h_ref[...] = h
        p_ref[...] = jnp.dot(h, wa_ref[...], preferred_element_type=jnp.float32)
        q_ref[...] = jnp.dot(h, wb_ref[...], preferred_element_type=jnp.float32)

    out = jax.ShapeDtypeStruct((n2, 2 * HID), jnp.float32)
    return pl.pallas_call(
        body,
        grid=(n2 // bs,),
        in_specs=[_rows(bs, x12.shape[1]), _full(We_p.shape), _full(be_t.shape),
                  _full(Wa_p.shape), _full(Wb_p.shape)],
        out_specs=[_rows(bs, 2 * HID)] * 3,
        out_shape=[out, out, out],
    )(x12, We_p, be_t, Wa_p, Wb_p)


def _tc_edge(esum2, ea0v, ea1v, Wc_p, b1_t, W2_p, b2_t, bs=2048):
    m2 = esum2.shape[0]

    def body(es_ref, ea0_ref, ea1_ref, wc_ref, b1_ref, w2_ref, b2_ref,
             out_ref):
        t0 = ea0_ref[...]
        t1 = ea1_ref[...]
        term = (t0[:, 0:1] * wc_ref[0:1, :] + t1[:, 0:1] * wc_ref[1:2, :]
                + t0[:, 1:2] * wc_ref[2:3, :] + t1[:, 1:2] * wc_ref[3:4, :])
        x = es_ref[...] + term + b1_ref[...]
        e = _silu(x)
        y = _silu(jnp.dot(e, w2_ref[...],
                          preferred_element_type=jnp.float32) + b2_ref[...])
        out_ref[...] = y

    return pl.pallas_call(
        body,
        grid=(m2 // bs,),
        in_specs=[_rows(bs, 2 * HID), _rows(bs, 2), _rows(bs, 2),
                  _full(Wc_p.shape), _full(b1_t.shape), _full(W2_p.shape),
                  _full(b2_t.shape)],
        out_specs=_rows(bs, 2 * HID),
        out_shape=jax.ShapeDtypeStruct((m2, 2 * HID), jnp.float32),
    )(esum2, ea0v, ea1v, Wc_p, b1_t, W2_p, b2_t)


def _tc_node(h2, agg2, W1_p, b1_t, W2_p, b2_t, Wa_p, Wb_p, bs=1000):
    n2 = h2.shape[0]

    def body(h_ref, a_ref, w1_ref, b1_ref, w2_ref, b2_ref, wa_ref, wb_ref,
             hn_ref, p_ref, q_ref):
        hin = jnp.concatenate([h_ref[...], a_ref[...]], axis=1)
        m = _silu(jnp.dot(hin, w1_ref[...],
                          preferred_element_type=jnp.float32) + b1_ref[...])
        m = jnp.dot(m, w2_ref[...], preferred_element_type=jnp.float32) + b2_ref[...]
        hn = h_ref[...] + m
        hn_ref[...] = hn
        p_ref[...] = jnp.dot(hn, wa_ref[...], preferred_element_type=jnp.float32)
        q_ref[...] = jnp.dot(hn, wb_ref[...], preferred_element_type=jnp.float32)

    out = jax.ShapeDtypeStruct((n2, 2 * HID), jnp.float32)
    return pl.pallas_call(
        body,
        grid=(n2 // bs,),
        in_specs=[_rows(bs, 2 * HID), _rows(bs, 2 * HID), _full(W1_p.shape),
                  _full(b1_t.shape), _full(W2_p.shape), _full(b2_t.shape),
                  _full(Wa_p.shape), _full(Wb_p.shape)],
        out_specs=[_rows(bs, 2 * HID)] * 3,
        out_shape=[out, out, out],
    )(h2, agg2, W1_p, b1_t, W2_p, b2_t, Wa_p, Wb_p)


def _tc_node_dec(h2, agg2, W1_p, b1_t, W2_p, b2_t, Wd1_p, bd1_t, Wd2_p, bd2_t,
                 bs=1000):
    n2 = h2.shape[0]

    def body(h_ref, a_ref, w1_ref, b1_ref, w2_ref, b2_ref,
             wd1_ref, bd1_ref, wd2_ref, bd2_ref, out_ref):
        hin = jnp.concatenate([h_ref[...], a_ref[...]], axis=1)
        m = _silu(jnp.dot(hin, w1_ref[...],
                          preferred_element_type=jnp.float32) + b1_ref[...])
        m = jnp.dot(m, w2_ref[...], preferred_element_type=jnp.float32) + b2_ref[...]
        hn = h_ref[...] + m
        t = _silu(jnp.dot(hn, wd1_ref[...],
                          preferred_element_type=jnp.float32) + bd1_ref[...])
        out_ref[...] = jnp.dot(t, wd2_ref[...],
                               preferred_element_type=jnp.float32) + bd2_ref[...]

    return pl.pallas_call(
        body,
        grid=(n2 // bs,),
        in_specs=[_rows(bs, 2 * HID), _rows(bs, 2 * HID), _full(W1_p.shape),
                  _full(b1_t.shape), _full(W2_p.shape), _full(b2_t.shape),
                  _full(Wd1_p.shape), _full(bd1_t.shape), _full(Wd2_p.shape),
                  _full(bd2_t.shape)],
        out_specs=_rows(bs, 6),
        out_shape=jax.ShapeDtypeStruct((n2, 6), jnp.float32),
    )(h2, agg2, W1_p, b1_t, W2_p, b2_t, Wd1_p, bd1_t, Wd2_p, bd2_t)


def _sc_gather_add(P, Q, row2d, col2d, e_pad):
    gpc = KG // G
    epw = e_pad // NW
    n_chunks = epw // KG
    nsb = n_chunks // IBG

    mesh = plsc.VectorSubcoreMesh(core_axis_name="c", subcore_axis_name="s",
                                  num_cores=NC, num_subcores=NS)

    @functools.partial(
        pl.kernel,
        out_type=jax.ShapeDtypeStruct((e_pad, HID), jnp.float32),
        mesh=mesh,
        scratch_types=[
            pltpu.VMEM((2, IBG, gpc, G), jnp.int32),
            pltpu.VMEM((2, IBG, gpc, G), jnp.int32),
            pltpu.VMEM((2, KG, HID), jnp.float32),
            pltpu.VMEM((2, KG, HID), jnp.float32),
            pltpu.SemaphoreType.DMA,
            pltpu.SemaphoreType.DMA,
            pltpu.SemaphoreType.DMA,
        ],
        compiler_params=pltpu.CompilerParams(use_tc_tiling_on_sc=False),
    )
    def k(p_hbm, q_hbm, row_hbm, col_hbm, out_hbm, idxr, idxc, S, D,
          semA, semB, semD):
        c = lax.axis_index("c")
        s = lax.axis_index("s")
        wid = s * NC + c
        base = wid * epw
        cbase = wid * n_chunks
        sems = (semA, semB)

        def idx_src(so):
            sl = pl.ds(cbase + so * IBG, IBG)
            return row_hbm.at[sl], col_hbm.at[sl]

        def fire_idx(so, p):
            r, cc = idx_src(so)
            pltpu.async_copy(r, idxr.at[p], semD)
            pltpu.async_copy(cc, idxc.at[p], semD)

        def wait_idx(so, p):
            r, cc = idx_src(so)
            pltpu.make_async_copy(r, idxr.at[p], semD).wait()
            pltpu.make_async_copy(cc, idxc.at[p], semD).wait()

        def fire(ci, kk, p, b):
            for g in range(gpc):
                pltpu.async_copy(p_hbm.at[idxr.at[p, kk, g]],
                                 S.at[b, pl.ds(g * G, G)], sems[b])
                pltpu.async_copy(q_hbm.at[idxc.at[p, kk, g]],
                                 D.at[b, pl.ds(g * G, G)], sems[b])

        def drain(kk, p, b):
            for g in range(gpc):
                pltpu.make_async_copy(p_hbm.at[idxr.at[p, kk, g]],
                                      S.at[b, pl.ds(g * G, G)],
                                      sems[b]).wait()
                pltpu.make_async_copy(q_hbm.at[idxc.at[p, kk, g]],
                                      D.at[b, pl.ds(g * G, G)],
                                      sems[b]).wait()

        fire_idx(0, 0)
        wait_idx(0, 0)
        fire(0, 0, 0, 0)

        def outer(oo, _):
            for sp in range(2):
                so = 2 * oo + sp
                p = sp
                for kk in range(IBG):
                    ci = so * IBG + kk
                    b = (sp + kk) % 2
                    if kk == 0:
                        @pl.when(so + 1 < nsb)
                        def _():
                            fire_idx(so + 1, 1 - p)
                    if kk + 1 < IBG:
                        fire(ci + 1, kk + 1, p, 1 - b)
                    else:
                        @pl.when(so + 1 < nsb)
                        def _():
                            wait_idx(so + 1, 1 - p)
                            fire(ci + 1, 0, 1 - p, 1 - b)
                    drain(kk, p, b)

                    def add_row(r, _):
                        for j in range(HID // LANES):
                            sl = pl.ds(j * LANES, LANES)
                            S[b, r, sl] = S[b, r, sl] + D[b, r, sl]
                        return 0

                    lax.fori_loop(0, KG, add_row, 0)
                    pltpu.sync_copy(S.at[b],
                                    out_hbm.at[pl.ds(base + ci * KG, KG)])
            return 0

        lax.fori_loop(0, nsb // 2, outer, 0)

    return k(P, Q, row2d, col2d)


def _sc_scatter_add(e2, row2d, n_nodes, e_pad):
    nhalf = n_nodes // 2
    stripe = 1568
    acc_rows = NS * stripe
    piece = 112
    npc = stripe // piece
    tail = nhalf - 15 * stripe - (npc - 1) * piece
    eps = e_pad // NS
    n_chunks = eps // KS
    nsb = n_chunks // IBS

    mesh = plsc.VectorSubcoreMesh(core_axis_name="c", subcore_axis_name="s",
                                  num_cores=NC, num_subcores=NS)

    @functools.partial(
        pl.kernel,
        out_type=jax.ShapeDtypeStruct((n_nodes, HID), jnp.float32),
        mesh=mesh,
        scratch_types=[
            pltpu.VMEM((2, IBS, G), jnp.int32),
            pltpu.VMEM((3, KS, HID), jnp.float32),
            pltpu.VMEM_SHARED((acc_rows, HID), jnp.float32),
            pltpu.SemaphoreType.DMA,
            pltpu.SemaphoreType.DMA,
            pltpu.SemaphoreType.DMA,
            pltpu.SemaphoreType.DMA,
        ],
        compiler_params=pltpu.CompilerParams(use_tc_tiling_on_sc=False),
    )
    def k(e_hbm, row_hbm, agg_hbm, idx, X, acc, semA, semB, semC, semD):
        c = lax.axis_index("c")
        s = lax.axis_index("s")
        half_base = c * nhalf
        sems = (semA, semB, semC)

        def zrow(r, _):
            for j in range(HID // LANES):
                X[0, r, pl.ds(j * LANES, LANES)] = jnp.zeros(
                    (LANES,), jnp.float32)
            return 0

        lax.fori_loop(0, piece, zrow, 0)
        for p in range(npc):
            pltpu.sync_copy(X.at[0, pl.ds(0, piece)],
                            acc.at[pl.ds(s * stripe + p * piece, piece)])
        plsc.subcore_barrier()

        gbase = s * n_chunks

        def fire_idx(so, p):
            pltpu.async_copy(row_hbm.at[pl.ds(gbase + so * IBS, IBS)],
                             idx.at[p], semD)

        def wait_idx(so, p):
            pltpu.make_async_copy(
                row_hbm.at[pl.ds(gbase + so * IBS, IBS)], idx.at[p],
                semD).wait()

        def fire_load(ci, b):
            pltpu.async_copy(e_hbm.at[pl.ds((gbase + ci) * KS, KS)],
                             X.at[b], sems[b])

        def wait_load(ci, b):
            pltpu.make_async_copy(
                e_hbm.at[pl.ds((gbase + ci) * KS, KS)], X.at[b],
                sems[b]).wait()

        def fire_scat(p, kk, b):
            pltpu.async_copy(X.at[b], acc.at[idx.at[p, kk]], sems[b],
                             add=True)

        def wait_scat(p, kk, b):
            pltpu.make_async_copy(X.at[b], acc.at[idx.at[p, kk]],
                                  sems[b]).wait()

        def transform(p, kk):
            for j in range(G // LANES):
                sl = pl.ds(j * LANES, LANES)
                v = idx[p, kk, sl] - half_base
                ok = (v >= 0) & (v < nhalf)
                idx[p, kk, sl] = jnp.where(ok, v, nhalf)

        def step(ci, so, p, kk, guard_first):
            b = kk % 3
            if guard_first and kk < 2:
                pass
            elif kk >= 2:
                wait_scat(p, kk - 2, (b + 1) % 3)
            else:
                wait_scat(1 - p, kk - 2 + IBS, (b + 1) % 3)
            if kk == 2 and not guard_first:
                @pl.when(so + 1 < nsb)
                def _():
                    fire_idx(so + 1, 1 - p)
            if guard_first and kk == 2:
                fire_idx(1, 1 - p)

            @pl.when(ci + 1 < n_chunks)
            def _():
                fire_load(ci + 1, (b + 1) % 3)

            wait_load(ci, b)
            transform(p, kk)
            fire_scat(p, kk, b)

        fire_idx(0, 0)
        wait_idx(0, 0)
        fire_load(0, 0)
        for kk in range(IBS):
            step(kk, 0, 0, kk, guard_first=True)

        def outer(oo, _):
            for sp in range(2):
                so = 1 + 2 * oo + sp
                p = (1 + sp) % 2
                if sp == 0:
                    wait_idx(so, p)
                for kk in range(IBS):
                    ci = so * IBS + kk
                    if sp == 1 and kk == 0:
                        wait_idx(so, p)
                    step(ci, so, p, kk, guard_first=False)
            return 0

        lax.fori_loop(0, (nsb - 1) // 2, outer, 0)
        wait_scat(0, IBS - 2, (IBS - 2) % 3)
        wait_scat(0, IBS - 1, (IBS - 1) % 3)
        plsc.subcore_barrier()

        start = s * stripe
        for p in range(npc):
            @pl.when((s < NS - 1) | (p < npc - 1))
            def _():
                pltpu.sync_copy(acc.at[pl.ds(start + p * piece, piece)],
                                X.at[0, pl.ds(0, piece)])
                pltpu.sync_copy(
                    X.at[0, pl.ds(0, piece)],
                    agg_hbm.at[pl.ds(half_base + start + p * piece, piece)])

        @pl.when(s == NS - 1)
        def _():
            o = start + (npc - 1) * piece
            pltpu.sync_copy(acc.at[pl.ds(o, tail)], X.at[0, pl.ds(0, tail)])
            pltpu.sync_copy(X.at[0, pl.ds(0, tail)],
                            agg_hbm.at[pl.ds(half_base + o, tail)])

    return k(e2, row2d)


def kernel(nodes, loc, edges, vel, edge_attr, params):
    n = loc.shape[0]
    e = edges.shape[1]
    row, col = edges[0], edges[1]

    e_pad = ((e + EQ - 1) // EQ) * EQ
    pad = e_pad - e
    assert e_pad // NW % (KG * IBG) == 0
    assert (e_pad // NW // KG // IBG) % 2 == 0
    assert e_pad // NS // KS % IBS == 0
    assert (e_pad // NS // KS // IBS) % 2 == 1

    x12 = jnp.concatenate([loc, vel], axis=1).reshape(n // 2, 12)
    row_g = jnp.concatenate(
        [row, jnp.zeros((pad,), jnp.int32)]).reshape(-1, KG // G, G)
    col_g = jnp.concatenate(
        [col, jnp.zeros((pad,), jnp.int32)]).reshape(-1, KG // G, G)
    row_s = jnp.concatenate(
        [row, jnp.full((pad,), n, jnp.int32)]).reshape(-1, G)
    eat = edge_attr.T
    zpad = jnp.zeros((pad,), jnp.float32)
    ea0v = jnp.concatenate([eat[0], zpad]).reshape(e_pad // 2, 2)
    ea1v = jnp.concatenate([eat[1], zpad]).reshape(e_pad // 2, 2)

    p = params
    NL = 4
    We_p = _bd(p['Wemb'])
    Wa = [p['We1_%d' % i][:HID] for i in range(NL)]
    Wb = [p['We1_%d' % i][HID:2 * HID] for i in range(NL)]
    Wc_p = [_bd(p['We1_%d' % i][2 * HID:]) for i in range(NL)]
    b1_t = [_bt(p['be1_%d' % i]) for i in range(NL)]
    W2_p = [_bd(p['We2_%d' % i]) for i in range(NL)]
    b2_t = [_bt(p['be2_%d' % i]) for i in range(NL)]
    Wn1_p = [jnp.concatenate([_bd(p['Wn1_%d' % i][:HID]),
                              _bd(p['Wn1_%d' % i][HID:])], axis=0)
             for i in range(NL)]
    bn1_t = [_bt(p['bn1_%d' % i]) for i in range(NL)]
    Wn2_p = [_bd(p['Wn2_%d' % i]) for i in range(NL)]
    bn2_t = [_bt(p['bn2_%d' % i]) for i in range(NL)]
    Wa_p = [_bd(w) for w in Wa]
    Wb_p = [_bd(w) for w in Wb]
    Wd1_p = _bd(p['Wd1'])
    bd1_t = _bt(p['bd1'])
    Wd2_p = _bd(p['Wd2'])
    bd2_t = _bt(p['bd2'])

    h2, P2, Q2 = _tc_embed(x12, We_p, _bt(p['bemb']), Wa_p[0], Wb_p[0])

    for i in range(NL):
        P = P2.reshape(n, HID)
        Q = Q2.reshape(n, HID)
        esum = _sc_gather_add(P, Q, row_g, col_g, e_pad)
        e2 = _tc_edge(esum.reshape(e_pad // 2, 2 * HID), ea0v, ea1v,
                      Wc_p[i], b1_t[i], W2_p[i], b2_t[i])
        agg = _sc_scatter_add(e2.reshape(e_pad, HID), row_s, n, e_pad)
        agg2 = agg.reshape(n // 2, 2 * HID)
        if i < NL - 1:
            h2, P2, Q2 = _tc_node(h2, agg2, Wn1_p[i], bn1_t[i], Wn2_p[i],
                                  bn2_t[i], Wa_p[i + 1], Wb_p[i + 1])
        else:
            out = _tc_node_dec(h2, agg2, Wn1_p[i], bn1_t[i], Wn2_p[i],
                               bn2_t[i], Wd1_p, bd1_t, Wd2_p, bd2_t)

    return out.reshape(n, 3)


# speedup vs baseline: 1.1082x; 1.1082x over previous
import functools

import jax
import jax.numpy as jnp
from jax import lax
from jax.experimental import pallas as pl
from jax.experimental.pallas import tpu as pltpu
from jax.experimental.pallas import tpu_sc as plsc

NC = 2
NS = 16
NW = NC * NS
LANES = 16

KG = 256
G = 128
KS = 128
HID = 64


def _silu(x):
    return x * jax.nn.sigmoid(x)


def _bd(W):
    Z = jnp.zeros_like(W)
    return jnp.concatenate(
        [jnp.concatenate([W, Z], axis=1), jnp.concatenate([Z, W], axis=1)],
        axis=0)


def _bt(b):
    return jnp.concatenate([b, b])[None, :]


def _full(spec_shape):
    return pl.BlockSpec(spec_shape, lambda i: tuple(0 for _ in spec_shape))


def _rows(bs, width):
    return pl.BlockSpec((bs, width), lambda i: (i, 0))


def _tc_embed(x12, We_p, be_t, Wa_p, Wb_p, bs=1000):
    n2 = x12.shape[0]

    def body(x_ref, we_ref, be_ref, wa_ref, wb_ref, h_ref, p_ref, q_ref):
        h = jnp.dot(x_ref[...], we_ref[...],
                    preferred_element_type=jnp.float32) + be_ref[...]
        h_ref[...] = h
        hb = h.astype(jnp.bfloat16)
        p_ref[...] = jnp.dot(hb, wa_ref[...], preferred_element_type=jnp.float32)
        q_ref[...] = jnp.dot(hb, wb_ref[...], preferred_element_type=jnp.float32)

    out = jax.ShapeDtypeStruct((n2, 2 * HID), jnp.float32)
    return pl.pallas_call(
        body,
        grid=(n2 // bs,),
        in_specs=[_rows(bs, x12.shape[1]), _full(We_p.shape), _full(be_t.shape),
                  _full(Wa_p.shape), _full(Wb_p.shape)],
        out_specs=[_rows(bs, 2 * HID)] * 3,
        out_shape=[out, out, out],
    )(x12, We_p, be_t, Wa_p, Wb_p)


def _tc_edge(esum2, ea0v, ea1v, Wc_p, b1_t, W2_p, b2_t, bs=2048):
    m2 = esum2.shape[0]

    def body(es_ref, ea0_ref, ea1_ref, wc_ref, b1_ref, w2_ref, b2_ref,
             out_ref):
        t0 = ea0_ref[...]
        t1 = ea1_ref[...]
        term = (t0[:, 0:1] * wc_ref[0:1, :] + t1[:, 0:1] * wc_ref[1:2, :]
                + t0[:, 1:2] * wc_ref[2:3, :] + t1[:, 1:2] * wc_ref[3:4, :])
        x = es_ref[...] + term + b1_ref[...]
        e = _silu(x).astype(jnp.bfloat16)
        y = _silu(jnp.dot(e, w2_ref[...],
                          preferred_element_type=jnp.float32) + b2_ref[...])
        out_ref[...] = y

    return pl.pallas_call(
        body,
        grid=(m2 // bs,),
        in_specs=[_rows(bs, 2 * HID), _rows(bs, 2), _rows(bs, 2),
                  _full(Wc_p.shape), _full(b1_t.shape), _full(W2_p.shape),
                  _full(b2_t.shape)],
        out_specs=_rows(bs, 2 * HID),
        out_shape=jax.ShapeDtypeStruct((m2, 2 * HID), jnp.float32),
    )(esum2, ea0v, ea1v, Wc_p, b1_t, W2_p, b2_t)


def _tc_node(h2, agg2, W1_p, b1_t, W2_p, b2_t, Wa_p, Wb_p, bs=1000):
    n2 = h2.shape[0]

    def body(h_ref, a_ref, w1_ref, b1_ref, w2_ref, b2_ref, wa_ref, wb_ref,
             hn_ref, p_ref, q_ref):
        hin = jnp.concatenate([h_ref[...], a_ref[...]],
                              axis=1).astype(jnp.bfloat16)
        m = _silu(jnp.dot(hin, w1_ref[...],
                          preferred_element_type=jnp.float32) + b1_ref[...])
        m = jnp.dot(m.astype(jnp.bfloat16), w2_ref[...],
                    preferred_element_type=jnp.float32) + b2_ref[...]
        hn = h_ref[...] + m
        hn_ref[...] = hn
        hb = hn.astype(jnp.bfloat16)
        p_ref[...] = jnp.dot(hb, wa_ref[...], preferred_element_type=jnp.float32)
        q_ref[...] = jnp.dot(hb, wb_ref[...], preferred_element_type=jnp.float32)

    out = jax.ShapeDtypeStruct((n2, 2 * HID), jnp.float32)
    return pl.pallas_call(
        body,
        grid=(n2 // bs,),
        in_specs=[_rows(bs, 2 * HID), _rows(bs, 2 * HID), _full(W1_p.shape),
                  _full(b1_t.shape), _full(W2_p.shape), _full(b2_t.shape),
                  _full(Wa_p.shape), _full(Wb_p.shape)],
        out_specs=[_rows(bs, 2 * HID)] * 3,
        out_shape=[out, out, out],
    )(h2, agg2, W1_p, b1_t, W2_p, b2_t, Wa_p, Wb_p)


def _tc_node_dec(h2, agg2, W1_p, b1_t, W2_p, b2_t, Wd1_p, bd1_t, Wd2_p, bd2_t,
                 bs=1000):
    n2 = h2.shape[0]

    def body(h_ref, a_ref, w1_ref, b1_ref, w2_ref, b2_ref,
             wd1_ref, bd1_ref, wd2_ref, bd2_ref, out_ref):
        hin = jnp.concatenate([h_ref[...], a_ref[...]],
                              axis=1).astype(jnp.bfloat16)
        m = _silu(jnp.dot(hin, w1_ref[...],
                          preferred_element_type=jnp.float32) + b1_ref[...])
        m = jnp.dot(m.astype(jnp.bfloat16), w2_ref[...],
                    preferred_element_type=jnp.float32) + b2_ref[...]
        hn = h_ref[...] + m
        t = _silu(jnp.dot(hn.astype(jnp.bfloat16), wd1_ref[...],
                          preferred_element_type=jnp.float32) + bd1_ref[...])
        out_ref[...] = jnp.dot(t, wd2_ref[...],
                               preferred_element_type=jnp.float32) + bd2_ref[...]

    return pl.pallas_call(
        body,
        grid=(n2 // bs,),
        in_specs=[_rows(bs, 2 * HID), _rows(bs, 2 * HID), _full(W1_p.shape),
                  _full(b1_t.shape), _full(W2_p.shape), _full(b2_t.shape),
                  _full(Wd1_p.shape), _full(bd1_t.shape), _full(Wd2_p.shape),
                  _full(bd2_t.shape)],
        out_specs=_rows(bs, 6),
        out_shape=jax.ShapeDtypeStruct((n2, 6), jnp.float32),
    )(h2, agg2, W1_p, b1_t, W2_p, b2_t, Wd1_p, bd1_t, Wd2_p, bd2_t)


def _sc_gather_add(P, Q, row2d, col2d, e_pad):
    n_chunks = e_pad // (NW * KG)
    gpc = KG // G
    epw = e_pad // NW

    mesh = plsc.VectorSubcoreMesh(core_axis_name="c", subcore_axis_name="s",
                                  num_cores=NC, num_subcores=NS)

    @functools.partial(
        pl.kernel,
        out_type=jax.ShapeDtypeStruct((e_pad, HID), jnp.float32),
        mesh=mesh,
        scratch_types=[
            pltpu.VMEM((2, gpc, G), jnp.int32),
            pltpu.VMEM((2, gpc, G), jnp.int32),
            pltpu.VMEM((2, KG, HID), jnp.float32),
            pltpu.VMEM((2, KG, HID), jnp.float32),
            pltpu.SemaphoreType.DMA,
            pltpu.SemaphoreType.DMA,
        ],
        compiler_params=pltpu.CompilerParams(use_tc_tiling_on_sc=False),
    )
    def k(p_hbm, q_hbm, row_hbm, col_hbm, out_hbm, idxr, idxc, S, D,
          semA, semB):
        c = lax.axis_index("c")
        s = lax.axis_index("s")
        wid = s * NC + c
        base = wid * epw
        gbase = wid * n_chunks
        sems = (semA, semB)

        def fire(ci, b):
            pltpu.sync_copy(row_hbm.at[gbase + ci], idxr.at[b])
            pltpu.sync_copy(col_hbm.at[gbase + ci], idxc.at[b])
            for g in range(gpc):
                pltpu.async_copy(p_hbm.at[idxr.at[b, g]],
                                 S.at[b, pl.ds(g * G, G)], sems[b])
                pltpu.async_copy(q_hbm.at[idxc.at[b, g]],
                                 D.at[b, pl.ds(g * G, G)], sems[b])

        def drain(b):
            for g in range(gpc):
                pltpu.make_async_copy(p_hbm.at[idxr.at[b, g]],
                                      S.at[b, pl.ds(g * G, G)],
                                      sems[b]).wait()
                pltpu.make_async_copy(q_hbm.at[idxc.at[b, g]],
                                      D.at[b, pl.ds(g * G, G)],
                                      sems[b]).wait()

        fire(0, 0)

        def outer(o, _):
            for b in range(2):
                ci = 2 * o + b

                @pl.when(ci + 1 < n_chunks)
                def _():
                    fire(ci + 1, 1 - b)

                drain(b)

                def add_row(r, _):
                    for j in range(HID // LANES):
                        sl = pl.ds(j * LANES, LANES)
                        S[b, r, sl] = S[b, r, sl] + D[b, r, sl]
                    return 0

                lax.fori_loop(0, KG, add_row, 0)
                pltpu.sync_copy(S.at[b],
                                out_hbm.at[pl.ds(base + ci * KG, KG)])
            return 0

        lax.fori_loop(0, n_chunks // 2, outer, 0)

    return k(P, Q, row2d, col2d)


def _sc_scatter_add(e2, row2d, n_nodes, e_pad):
    nhalf = n_nodes // 2
    stripe = 1568
    acc_rows = NS * stripe
    piece = 112
    npc = stripe // piece
    tail = nhalf - 15 * stripe - (npc - 1) * piece
    eps = e_pad // NS
    n_chunks = eps // KS

    mesh = plsc.VectorSubcoreMesh(core_axis_name="c", subcore_axis_name="s",
                                  num_cores=NC, num_subcores=NS)

    @functools.partial(
        pl.kernel,
        out_type=jax.ShapeDtypeStruct((n_nodes, HID), jnp.float32),
        mesh=mesh,
        scratch_types=[
            pltpu.VMEM((3, 1, G), jnp.int32),
            pltpu.VMEM((3, KS, HID), jnp.float32),
            pltpu.VMEM_SHARED((acc_rows, HID), jnp.float32),
            pltpu.SemaphoreType.DMA,
            pltpu.SemaphoreType.DMA,
            pltpu.SemaphoreType.DMA,
        ],
        compiler_params=pltpu.CompilerParams(use_tc_tiling_on_sc=False),
    )
    def k(e_hbm, row_hbm, agg_hbm, idx, X, acc, semA, semB, semC):
        c = lax.axis_index("c")
        s = lax.axis_index("s")
        half_base = c * nhalf
        sems = (semA, semB, semC)

        def zrow(r, _):
            for j in range(HID // LANES):
                X[0, r, pl.ds(j * LANES, LANES)] = jnp.zeros(
                    (LANES,), jnp.float32)
            return 0

        lax.fori_loop(0, piece, zrow, 0)
        for p in range(npc):
            pltpu.sync_copy(X.at[0, pl.ds(0, piece)],
                            acc.at[pl.ds(s * stripe + p * piece, piece)])
        plsc.subcore_barrier()

        gbase = s * n_chunks

        def fire_load(ci, b):
            pltpu.sync_copy(row_hbm.at[gbase + ci], idx.at[b])
            pltpu.async_copy(e_hbm.at[pl.ds((gbase + ci) * KS, KS)],
                             X.at[b], sems[b])

        def wait_load(ci, b):
            pltpu.make_async_copy(
                e_hbm.at[pl.ds((gbase + ci) * KS, KS)], X.at[b],
                sems[b]).wait()

        def fire_scat(b):
            pltpu.async_copy(X.at[b], acc.at[idx.at[b, 0]], sems[b],
                             add=True)

        def wait_scat(b):
            pltpu.make_async_copy(X.at[b], acc.at[idx.at[b, 0]],
                                  sems[b]).wait()

        def transform(b):
            for j in range(G // LANES):
                sl = pl.ds(j * LANES, LANES)
                v = idx[b, 0, sl] - half_base
                ok = (v >= 0) & (v < nhalf)
                idx[b, 0, sl] = jnp.where(ok, v, nhalf)

        fire_load(0, 0)

        def outer(o, _):
            for b in range(3):
                ci = 3 * o + b

                @pl.when(ci >= 2)
                def _():
                    wait_scat((b + 1) % 3)

                @pl.when(ci + 1 < n_chunks)
                def _():
                    fire_load(ci + 1, (b + 1) % 3)

                wait_load(ci, b)
                transform(b)
                fire_scat(b)
            return 0

        n_main = n_chunks // 3
        lax.fori_loop(0, n_main, outer, 0)
        for ci in range(3 * n_main, n_chunks):
            b = ci % 3
            wait_scat((b + 1) % 3)
            if ci + 1 < n_chunks:
                fire_load(ci + 1, (b + 1) % 3)
            wait_load(ci, b)
            transform(b)
            fire_scat(b)
        for ci in range(n_chunks - 2, n_chunks):
            wait_scat(ci % 3)
        plsc.subcore_barrier()

        start = s * stripe
        for p in range(npc):
            @pl.when((s < NS - 1) | (p < npc - 1))
            def _():
                pltpu.sync_copy(acc.at[pl.ds(start + p * piece, piece)],
                                X.at[0, pl.ds(0, piece)])
                pltpu.sync_copy(
                    X.at[0, pl.ds(0, piece)],
                    agg_hbm.at[pl.ds(half_base + start + p * piece, piece)])

        @pl.when(s == NS - 1)
        def _():
            o = start + (npc - 1) * piece
            pltpu.sync_copy(acc.at[pl.ds(o, tail)], X.at[0, pl.ds(0, tail)])
            pltpu.sync_copy(X.at[0, pl.ds(0, tail)],
                            agg_hbm.at[pl.ds(half_base + o, tail)])

    return k(e2, row2d)


def kernel(nodes, loc, edges, vel, edge_attr, params):
    n = loc.shape[0]
    e = edges.shape[1]
    row, col = edges[0], edges[1]

    chunk_all = NW * KG
    e_pad = ((e + chunk_all - 1) // chunk_all) * chunk_all
    pad = e_pad - e

    x12 = jnp.concatenate([loc, vel], axis=1).reshape(n // 2, 12)
    row_g = jnp.concatenate(
        [row, jnp.zeros((pad,), jnp.int32)]).reshape(-1, KG // G, G)
    col_g = jnp.concatenate(
        [col, jnp.zeros((pad,), jnp.int32)]).reshape(-1, KG // G, G)
    row_s = jnp.concatenate(
        [row, jnp.full((pad,), n, jnp.int32)]).reshape(-1, KS // G, G)
    eat = edge_attr.T
    zpad = jnp.zeros((pad,), jnp.float32)
    ea0v = jnp.concatenate([eat[0], zpad]).reshape(e_pad // 2, 2)
    ea1v = jnp.concatenate([eat[1], zpad]).reshape(e_pad // 2, 2)

    p = params
    NL = 4
    We_p = _bd(p['Wemb'])
    Wa = [p['We1_%d' % i][:HID] for i in range(NL)]
    Wb = [p['We1_%d' % i][HID:2 * HID] for i in range(NL)]
    Wc_p = [_bd(p['We1_%d' % i][2 * HID:]) for i in range(NL)]
    b1_t = [_bt(p['be1_%d' % i]) for i in range(NL)]
    W2_p = [_bd(p['We2_%d' % i]).astype(jnp.bfloat16) for i in range(NL)]
    b2_t = [_bt(p['be2_%d' % i]) for i in range(NL)]
    bf = jnp.bfloat16
    Wn1_p = [jnp.concatenate([_bd(p['Wn1_%d' % i][:HID]),
                              _bd(p['Wn1_%d' % i][HID:])],
                             axis=0).astype(bf)
             for i in range(NL)]
    bn1_t = [_bt(p['bn1_%d' % i]) for i in range(NL)]
    Wn2_p = [_bd(p['Wn2_%d' % i]).astype(bf) for i in range(NL)]
    bn2_t = [_bt(p['bn2_%d' % i]) for i in range(NL)]
    Wa_p = [_bd(w).astype(bf) for w in Wa]
    Wb_p = [_bd(w).astype(bf) for w in Wb]
    Wd1_p = _bd(p['Wd1']).astype(bf)
    bd1_t = _bt(p['bd1'])
    Wd2_p = _bd(p['Wd2'])
    bd2_t = _bt(p['bd2'])

    h2, P2, Q2 = _tc_embed(x12, We_p, _bt(p['bemb']), Wa_p[0], Wb_p[0])

    for i in range(NL):
        P = P2.reshape(n, HID)
        Q = Q2.reshape(n, HID)
        esum = _sc_gather_add(P, Q, row_g, col_g, e_pad)
        e2 = _tc_edge(esum.reshape(e_pad // 2, 2 * HID), ea0v, ea1v,
                      Wc_p[i], b1_t[i], W2_p[i], b2_t[i])
        agg = _sc_scatter_add(e2.reshape(e_pad, HID), row_s, n, e_pad)
        agg2 = agg.reshape(n // 2, 2 * HID)
        if i < NL - 1:
            h2, P2, Q2 = _tc_node(h2, agg2, Wn1_p[i], bn1_t[i], Wn2_p[i],
                                  bn2_t[i], Wa_p[i + 1], Wb_p[i + 1])
        else:
            out = _tc_node_dec(h2, agg2, Wn1_p[i], bn1_t[i], Wn2_p[i],
                               bn2_t[i], Wd1_p, bd1_t, Wd2_p, bd2_t)

    return out.reshape(n, 3)
